# Initial kernel scaffold; baseline (speedup 1.0000x reference)
import jax
import jax.numpy as jnp
from jax.experimental import pallas as pl


def kernel(logits):
    raise NotImplementedError("write your pallas kernel here")


# speedup vs baseline: 1.3166x; 1.3166x over previous
import functools

import jax
import jax.numpy as jnp
from jax import lax
from jax.experimental import pallas as pl
from jax.experimental.pallas import tpu as pltpu
from jax.experimental.pallas import tpu_sc as plsc

L = 16
V = 8192
NROWS = 512
NWORKERS = 32
ROWS_PER = NROWS // NWORKERS
NCHUNK = V // L


def _body(x_hbm, out_hbm, xbuf, obuf):
    cid = lax.axis_index("c")
    sid = lax.axis_index("s")
    wid = sid * 2 + cid
    base = wid * ROWS_PER

    lanes = lax.iota(jnp.int32, L)
    zeros = jnp.zeros((L,), jnp.float32)
    ones = jnp.ones((L,), jnp.float32)
    mask0 = lanes == 0

    def zbody(j, c):
        obuf[pl.ds(j * L, L)] = zeros
        return c

    lax.fori_loop(0, NCHUNK, zbody, 0)

    def row_step(r, c):
        row = base + r
        pltpu.sync_copy(x_hbm.at[row], xbuf)

        def cbody(j, carry):
            m, idx = carry
            x = xbuf[pl.ds(j * L, L)]
            cond = x > m
            m2 = jnp.where(cond, x, m)
            idx2 = jnp.where(cond, j * L + lanes, idx)
            return (m2, idx2)

        m0 = jnp.full((L,), -jnp.inf, jnp.float32)
        i0 = jnp.zeros((L,), jnp.int32)
        m, idx = lax.fori_loop(0, NCHUNK, cbody, (m0, i0))

        gm = m[0]
        gi = idx[0]
        for k in range(1, L):
            mv = m[k]
            iv = idx[k]
            take = (mv > gm) | ((mv == gm) & (iv < gi))
            gm = jnp.where(take, mv, gm)
            gi = jnp.where(take, iv, gi)

        idxv = jnp.full((L,), gi, jnp.int32)
        plsc.store_scatter(obuf, [idxv], ones, mask=mask0)
        pltpu.sync_copy(obuf, out_hbm.at[row])
        plsc.store_scatter(obuf, [idxv], zeros, mask=mask0)
        return c

    lax.fori_loop(0, ROWS_PER, row_step, 0)


@jax.jit
def kernel(logits):
    B, S, _ = logits.shape
    x = logits.reshape(NROWS, V)
    out = pl.kernel(
        _body,
        out_type=jax.ShapeDtypeStruct((NROWS, V), jnp.float32),
        mesh=plsc.VectorSubcoreMesh(core_axis_name="c", subcore_axis_name="s"),
        compiler_params=pltpu.CompilerParams(needs_layout_passes=False),
        scratch_types=[
            pltpu.VMEM((V,), jnp.float32),
            pltpu.VMEM((V,), jnp.float32),
        ],
    )(x)
    return out.reshape(B, S, V)


# speedup vs baseline: 2.3463x; 1.7821x over previous
import jax
import jax.numpy as jnp
from jax import lax
from jax.experimental import pallas as pl
from jax.experimental.pallas import tpu as pltpu
from jax.experimental.pallas import tpu_sc as plsc

L = 16
V = 8192
NROWS = 512
NWORKERS = 32
ROWS_PER = NROWS // NWORKERS
NCHAIN = 4
NSTEP = V // (L * NCHAIN)


def _merge(ma, ia, mb, ib):
    take = (mb > ma) | ((mb == ma) & (ib < ia))
    return jnp.where(take, mb, ma), jnp.where(take, ib, ia)


def _argmax_row(xbuf, lanes):
    ms = [jnp.full((L,), -jnp.inf, jnp.float32) for _ in range(NCHAIN)]
    iis = [jnp.zeros((L,), jnp.int32) for _ in range(NCHAIN)]
    curs = [lanes + L * k for k in range(NCHAIN)]

    def cbody(j, carry):
        ms, iis, curs = carry
        base = j * (L * NCHAIN)
        nms, nis, ncurs = [], [], []
        for k in range(NCHAIN):
            x = xbuf[pl.ds(base + k * L, L)]
            cond = x > ms[k]
            nms.append(jnp.where(cond, x, ms[k]))
            nis.append(jnp.where(cond, curs[k], iis[k]))
            ncurs.append(curs[k] + L * NCHAIN)
        return (tuple(nms), tuple(nis), tuple(ncurs))

    ms, iis, _ = lax.fori_loop(0, NSTEP, cbody, (tuple(ms), tuple(iis), tuple(curs)))

    m01, i01 = _merge(ms[0], iis[0], ms[1], iis[1])
    m23, i23 = _merge(ms[2], iis[2], ms[3], iis[3])
    m, idx = _merge(m01, i01, m23, i23)

    gm = m[0]
    gi = idx[0]
    for k in range(1, L):
        mv = m[k]
        iv = idx[k]
        take = (mv > gm) | ((mv == gm) & (iv < gi))
        gm = jnp.where(take, mv, gm)
        gi = jnp.where(take, iv, gi)
    return gi


def _body(x_hbm, out_hbm, xb0, xb1, ob0, ob1, si0, si1, so0, so1):
    cid = lax.axis_index("c")
    sid = lax.axis_index("s")
    wid = sid * 2 + cid
    base = wid * ROWS_PER

    xbufs = (xb0, xb1)
    obufs = (ob0, ob1)
    sins = (si0, si1)
    souts = (so0, so1)

    lanes = lax.iota(jnp.int32, L)
    zeros = jnp.zeros((L,), jnp.float32)
    ones = jnp.ones((L,), jnp.float32)
    mask0 = lanes == 0

    def zbody(j, c):
        ob0[pl.ds(j * L, L)] = zeros
        ob1[pl.ds(j * L, L)] = zeros
        return c

    lax.fori_loop(0, V // L, zbody, 0)

    pltpu.async_copy(x_hbm.at[base], xb0, si0)

    prev_idxv = [None, None]
    for r in range(ROWS_PER):
        slot = r % 2
        pltpu.make_async_copy(x_hbm.at[base + r], xbufs[slot], sins[slot]).wait()
        if r + 1 < ROWS_PER:
            pltpu.async_copy(
                x_hbm.at[base + r + 1], xbufs[1 - slot], sins[1 - slot]
            )

        gi = _argmax_row(xbufs[slot], lanes)
        idxv = jnp.full((L,), gi, jnp.int32)

        if r >= 2:
            pltpu.make_async_copy(
                obufs[slot], out_hbm.at[base + r - 2], souts[slot]
            ).wait()
            plsc.store_scatter(obufs[slot], [prev_idxv[slot]], zeros, mask=mask0)

        plsc.store_scatter(obufs[slot], [idxv], ones, mask=mask0)
        pltpu.async_copy(obufs[slot], out_hbm.at[base + r], souts[slot])
        prev_idxv[slot] = idxv

    pltpu.make_async_copy(ob0, out_hbm.at[base + ROWS_PER - 2], so0).wait()
    pltpu.make_async_copy(ob1, out_hbm.at[base + ROWS_PER - 1], so1).wait()


@jax.jit
def kernel(logits):
    B, S, _ = logits.shape
    x = logits.reshape(NROWS, V)
    out = pl.kernel(
        _body,
        out_type=jax.ShapeDtypeStruct((NROWS, V), jnp.float32),
        mesh=plsc.VectorSubcoreMesh(core_axis_name="c", subcore_axis_name="s"),
        compiler_params=pltpu.CompilerParams(needs_layout_passes=False),
        scratch_types=[
            pltpu.VMEM((V,), jnp.float32),
            pltpu.VMEM((V,), jnp.float32),
            pltpu.VMEM((V,), jnp.float32),
            pltpu.VMEM((V,), jnp.float32),
            pltpu.SemaphoreType.DMA,
            pltpu.SemaphoreType.DMA,
            pltpu.SemaphoreType.DMA,
            pltpu.SemaphoreType.DMA,
        ],
    )(x)
    return out.reshape(B, S, V)
